# baseline (speedup 1.0000x reference)
import functools

import jax
import jax.numpy as jnp
from jax import lax
from jax.experimental import pallas as pl
from jax.experimental.pallas import tpu as pltpu
from jax.experimental.pallas import tpu_sc as plsc

_NUM_CORES = 2
_NUM_SUBCORES = 16
_NW = _NUM_CORES * _NUM_SUBCORES

_B = 16384
_D = 32
_L = 16
_TL = 128

_SH = 32768
_CH = 1024
_LANES_PAD = 1000064
_LAST_START = 999040
_CAP = 736
_NG = 47


@functools.partial(
    pl.kernel,
    mesh=plsc.VectorSubcoreMesh(core_axis_name="c", subcore_axis_name="s"),
    out_type=jax.ShapeDtypeStruct((_B, _TL), jnp.float32),
    scratch_types=[
        pltpu.VMEM((_B + _L,), jnp.int32),
        pltpu.VMEM((_CAP + 2 * _L,), jnp.int32),
        pltpu.VMEM((_CAP + 2 * _L,), jnp.int32),
        pltpu.VMEM((2, _D, _CH), jnp.float32),
        pltpu.VMEM((4, _L, _TL), jnp.float32),
        pltpu.SemaphoreType.DMA,
        pltpu.SemaphoreType.DMA,
    ],
    compiler_params=pltpu.CompilerParams(needs_layout_passes=False),
)
def _gather_stream(
    vt_hbm, idx_hbm, out_hbm, idx_v, ml_i, ml_k, chunks, rowbufs,
    sem_s, sem_w,
):
    cid = lax.axis_index("c")
    sid = lax.axis_index("s")
    wid = cid * _NUM_SUBCORES + sid
    iota = lax.iota(jnp.int32, _L)

    pltpu.sync_copy(idx_hbm, idx_v.at[pl.ds(0, _B)])

    big = jnp.full((_L,), jnp.int32(0x7FFFFFF0), jnp.int32)

    def sent(g, _):
        ml_i[pl.ds(g * _L, _L)] = big
        ml_k[pl.ds(g * _L, _L)] = jnp.zeros((_L,), jnp.int32)
        return 0

    lax.fori_loop(0, (_CAP + 2 * _L) // _L, sent, 0)

    wid_v = jnp.full((_L,), wid, jnp.int32)

    def binb(g, cnt):
        vec = idx_v[pl.ds(g * _L, _L)]
        kvec = iota + g * _L
        m = lax.shift_right_logical(vec, 15) == wid_v
        pref = plsc.cumsum(m.astype(jnp.int32))
        dest = jnp.where(m, cnt + pref - 1, _CAP + _L)
        plsc.store_scatter(ml_i, [dest], vec)
        plsc.store_scatter(ml_k, [dest], kvec)
        return lax.min(cnt + pref[_L - 1], _CAP)

    lax.fori_loop(0, _B // _L, binb, 0)

    wbase = wid * _SH
    nch = lax.max(0, lax.min(_SH // _CH, (_LANES_PAD - wbase + _CH - 1) >> 10))

    def chunk_start(j):
        return pl.multiple_of(lax.min(wbase + j * _CH, _LAST_START), 128)

    def fire(j):
        pltpu.make_async_copy(
            vt_hbm.at[:, pl.ds(chunk_start(j), _CH)],
            chunks.at[j % 2],
            sem_s,
        ).start()

    @pl.when(nch > 0)
    def _():
        fire(0)

    @pl.when(nch > 1)
    def _():
        fire(1)

    def per_chunk(j, sc):
        pltpu.make_async_copy(
            vt_hbm.at[:, pl.ds(0, _CH)], chunks.at[0], sem_s
        ).wait()
        st = chunk_start(j)
        st_v = jnp.full((_L,), st, jnp.int32)
        slot_v = jnp.full((_L,), j % 2, jnp.int32)

        def scan_g(g, sc_in):
            ivec = ml_i[pl.ds(g * _L, _L)]
            kvec = ml_k[pl.ds(g * _L, _L)]
            m = jnp.logical_and(ivec >= st_v, ivec < st_v + _CH)
            npop = plsc.all_reduce_population_count(m)[0]

            def hit(sc2):
                f = plsc.all_reduce_ffs(m)
                i_first = plsc.load_gather(ml_i, [g * _L + f])
                k_first = plsc.load_gather(ml_k, [g * _L + f])
                lane = jnp.where(m, ivec - st_v, i_first - st_v)
                keff = jnp.where(m, kvec, k_first)
                rb = sc2 % 4

                @pl.when(sc2 >= 4)
                def _():
                    pltpu.make_async_copy(
                        rowbufs.at[0], out_hbm.at[pl.ds(0, _L), :], sem_w
                    ).wait()

                rb_v = jnp.full((_L,), rb, jnp.int32)
                for jj in range(_D):
                    jj_v = jnp.full((_L,), jj, jnp.int32)
                    v = plsc.load_gather(chunks, [slot_v, jj_v, lane])
                    plsc.store_scatter(rowbufs, [rb_v, iota, jj_v], v)
                pltpu.make_async_copy(
                    rowbufs.at[rb], out_hbm.at[keff], sem_w
                ).start()
                return sc2 + 1

            return lax.cond(npop > 0, hit, lambda sc2: sc2, sc_in)

        sc_out = lax.fori_loop(0, _NG, scan_g, sc)

        @pl.when(j + 2 < nch)
        def _():
            fire(j + 2)

        return sc_out

    scat = lax.fori_loop(0, nch, per_chunk, 0)

    def drain(_, __):
        pltpu.make_async_copy(
            rowbufs.at[0], out_hbm.at[pl.ds(0, _L), :], sem_w
        ).wait()
        return 0

    lax.fori_loop(0, lax.min(scat, 4), drain, 0)


def kernel(indices, values):
    padded = _gather_stream(values.T, indices.astype(jnp.int32))
    return padded[:, :_D]


# speedup vs baseline: 2.0260x; 2.0260x over previous
import functools

import jax
import jax.numpy as jnp
from jax import lax
from jax.experimental import pallas as pl
from jax.experimental.pallas import tpu as pltpu
from jax.experimental.pallas import tpu_sc as plsc

_NUM_CORES = 2
_NUM_SUBCORES = 16
_NW = _NUM_CORES * _NUM_SUBCORES

_B = 16384
_D = 32
_BPW = _B // _NW
_L = 16
_TL = 128
_R = 16


@functools.partial(
    pl.kernel,
    mesh=plsc.VectorSubcoreMesh(core_axis_name="c", subcore_axis_name="s"),
    out_type=jax.ShapeDtypeStruct((_D, _B), jnp.float32),
    scratch_types=[
        pltpu.VMEM((_BPW + _L,), jnp.int32),
        pltpu.VMEM((_R, _D, _TL), jnp.float32),
        pltpu.VMEM((_D, _BPW), jnp.float32),
        pltpu.SemaphoreType.DMA,
    ],
    compiler_params=pltpu.CompilerParams(needs_layout_passes=False),
)
def _gather_cols(vt_hbm, idx_hbm, out_hbm, idx_s, slabs, cols_v, sem):
    wid = lax.axis_index("s") * _NUM_CORES + lax.axis_index("c")
    base = pl.multiple_of(wid * _BPW, _BPW)
    pltpu.sync_copy(idx_hbm.at[pl.ds(base, _BPW)], idx_s.at[pl.ds(0, _BPW)])

    def idx_at(k):
        return idx_s[pl.ds(k, _L)][0]

    def fetch(k, r):
        start = pl.multiple_of(
            lax.shift_right_logical(idx_at(k), 7) * _TL, _TL
        )
        pltpu.make_async_copy(
            vt_hbm.at[:, pl.ds(start, _TL)],
            slabs.at[r],
            sem,
        ).start()

    for r in range(_R):
        fetch(r, r)

    j_lo = lax.iota(jnp.int32, _L)
    j_hi = j_lo + _L

    def outer(it, _):
        k0 = it * _R
        for r in range(_R):
            k = k0 + r
            pltpu.make_async_copy(
                vt_hbm.at[:, pl.ds(0, _TL)], slabs.at[r], sem
            ).wait()
            lane = jnp.full((_L,), idx_at(k) & (_TL - 1), jnp.int32)
            r_vec = jnp.full((_L,), r, jnp.int32)
            k_vec = jnp.full((_L,), k, jnp.int32)
            v0 = plsc.load_gather(slabs, [r_vec, j_lo, lane])
            v1 = plsc.load_gather(slabs, [r_vec, j_hi, lane])
            plsc.store_scatter(cols_v, [j_lo, k_vec], v0)
            plsc.store_scatter(cols_v, [j_hi, k_vec], v1)

            @pl.when(k + _R < _BPW)
            def _():
                fetch(k + _R, r)

        return 0

    lax.fori_loop(0, _BPW // _R, outer, 0)
    pltpu.sync_copy(cols_v, out_hbm.at[:, pl.ds(base, _BPW)])


def kernel(indices, values):
    out_t = _gather_cols(values.T, indices.astype(jnp.int32))
    return out_t.T
